# baseline (speedup 1.0000x reference)
import functools

import jax
import jax.numpy as jnp
from jax import lax
from jax.experimental import pallas as pl
from jax.experimental.pallas import tpu as pltpu
from jax.experimental.pallas import tpu_sc as plsc

_NC = 2
_NS = 16
_NW = _NC * _NS

_CHUNK = 128
_GROUP = 8


@jax.jit
def _embed_lookup(x_flat, embedding):
    B = x_flat.shape[0]
    _, D = embedding.shape
    b_per_w = B // _NW
    n_chunks = b_per_w // _CHUNK
    n_groups = n_chunks // _GROUP
    assert b_per_w * _NW == B and n_chunks * _CHUNK == b_per_w
    assert n_groups * _GROUP == n_chunks
    idx3 = x_flat.reshape(_NW, n_chunks, _CHUNK)

    mesh = plsc.VectorSubcoreMesh(core_axis_name="c", subcore_axis_name="s")

    @functools.partial(
        pl.kernel,
        mesh=mesh,
        out_type=jax.ShapeDtypeStruct((B, D), jnp.float32),
        scratch_types=[
            pltpu.VMEM((n_chunks, _CHUNK), jnp.int32),
            pltpu.VMEM((_GROUP, _CHUNK, D), jnp.float32),
            pltpu.SemaphoreType.DMA,
            pltpu.SemaphoreType.DMA,
        ],
        compiler_params=pltpu.CompilerParams(use_tc_tiling_on_sc=False),
    )
    def gather_kernel(idx_hbm, table_hbm, out_hbm, idx_v, rows_v, gsem, wsem):
        wid = lax.axis_index("s") * _NC + lax.axis_index("c")
        base = wid * b_per_w
        pltpu.sync_copy(idx_hbm.at[wid], idx_v)

        @pl.loop(0, n_groups)
        def _group(g):
            j0 = g * _GROUP
            gathers = []
            for b in range(_GROUP):
                gathers.append(
                    pltpu.async_copy(
                        table_hbm.at[idx_v.at[j0 + b]], rows_v.at[b], gsem
                    )
                )
            writes = []
            for b in range(_GROUP):
                gathers[b].wait()
                writes.append(
                    pltpu.async_copy(
                        rows_v.at[b],
                        out_hbm.at[pl.ds(base + (j0 + b) * _CHUNK, _CHUNK)],
                        wsem,
                    )
                )
            for b in range(_GROUP):
                writes[b].wait()

    return gather_kernel(idx3, embedding)


def kernel(x, embedding):
    S0, S1 = x.shape
    out = _embed_lookup(x.reshape(S0 * S1), embedding)
    return (out.reshape(S0, S1, embedding.shape[1]), None)
